# baseline (speedup 1.0000x reference)
import functools

import jax
import jax.numpy as jnp
import numpy as np
from jax.experimental import pallas as pl

NUM_HEADS = 8
HDP = 128
_SIGN = np.int32(-2**31)


def _pool_matrix(hw: int) -> np.ndarray:
    p = np.zeros((hw * hw, hw * hw), np.float32)
    idx = np.arange(hw)
    for k in (3, 5, 7):
        a = (np.abs(idx[:, None] - idx[None, :]) <= k // 2).astype(np.float32)
        p += np.kron(a, a) / float(k * k)
    return p


def _stage1_body(y_ref, x_ref, pm_ref, wkT_ref, wvT_ref, wqT_ref,
                 lnw_ref, lnb_ref, kT_ref, vT_ref, qT_ref):
    yb = y_ref[0]
    y_hi = yb.astype(jnp.bfloat16)
    y_lo = (yb - y_hi.astype(jnp.float32)).astype(jnp.bfloat16)
    pmat = pm_ref[...]
    p_hi = pmat.astype(jnp.bfloat16)
    p_lo = (pmat - p_hi.astype(jnp.float32)).astype(jnp.bfloat16)
    ysT = (jnp.dot(y_hi, p_hi, preferred_element_type=jnp.float32)
           + jnp.dot(y_hi, p_lo, preferred_element_type=jnp.float32)
           + jnp.dot(y_lo, p_hi, preferred_element_type=jnp.float32))
    mu = jnp.mean(ysT, axis=0, keepdims=True)
    var = jnp.mean((ysT - mu) ** 2, axis=0, keepdims=True)
    ynT = (ysT - mu) / jnp.sqrt(var + 1e-5) * lnw_ref[...] + lnb_ref[...]
    ynT = ynT.astype(jnp.bfloat16)
    xT = x_ref[0].astype(jnp.bfloat16)
    kT_ref[0] = jnp.dot(wkT_ref[...], ynT,
                        preferred_element_type=jnp.float32).astype(jnp.bfloat16)
    vT_ref[0] = jnp.dot(wvT_ref[...], ynT,
                        preferred_element_type=jnp.float32).astype(jnp.bfloat16)
    qT_ref[0] = jnp.dot(wqT_ref[...], xT,
                        preferred_element_type=jnp.float32).astype(jnp.bfloat16)


def _select_thr(keys, k16, kk):
    rows, cols = keys.shape
    p16 = jnp.zeros((1, cols), jnp.int16)
    negkk = np.int16(-kk)
    for b in range(14, -1, -1):
        cand = p16 | np.int16(1 << b)
        d = jnp.where(k16 >= cand, np.int16(-1), np.int16(0))
        h = rows
        while h > 16:
            h //= 2
            d = d[:h] + d[h:2 * h]
        s = jnp.sum(d.astype(jnp.int32), axis=0, keepdims=True)
        p16 = jnp.where(s.astype(jnp.int16) <= negkk, cand, p16)
    p = p16.astype(jnp.int32) << 15
    for b in range(14, 9, -1):
        cand = p | np.int32(1 << b)
        cnt = jnp.sum((keys >= cand).astype(jnp.int32), axis=0, keepdims=True)
        p = jnp.where(cnt >= kk, cand, p)
    return p


def _attn_body(qT_ref, kT_ref, vT_ref, wpT_ref, bp_ref, a1_ref, a2_ref,
               res_ref, *, kk1, kk2, scale):
    h = pl.program_id(1)
    kT = kT_ref[0]
    qT = qT_ref[0]
    sT = jax.lax.dot_general(kT, qT, (((0,), (0,)), ((), ())),
                             preferred_element_type=jnp.float32) * scale
    m = jnp.max(sT, axis=0, keepdims=True)
    e = jnp.exp(sT - m)
    keys = jax.lax.bitcast_convert_type(e, jnp.int32)
    k16 = (keys >> 15).astype(jnp.int16)
    p1 = _select_thr(keys, k16, kk1)
    p2 = _select_thr(keys, k16, kk2)
    m1 = keys >= p1
    m2 = keys >= p2
    zero = jnp.float32(0.0)
    s1 = jnp.sum(jnp.where(m1, e, zero), axis=0, keepdims=True)
    s2 = jnp.sum(jnp.where(m2, e, zero), axis=0, keepdims=True)
    a1 = a1_ref[0, 0]
    a2 = a2_ref[0, 0]
    w = jnp.where(m1, e * (a1 / s1), zero) + jnp.where(m2, e * (a2 / s2),
                                                       zero)
    outT = jnp.dot(vT_ref[0], w.astype(jnp.bfloat16),
                   preferred_element_type=jnp.float32)
    contrib = jnp.dot(wpT_ref[0].astype(jnp.bfloat16),
                      outT.astype(jnp.bfloat16),
                      preferred_element_type=jnp.float32)

    @pl.when(h == 0)
    def _init():
        res_ref[0] = contrib + bp_ref[...]

    @pl.when(h != 0)
    def _acc():
        res_ref[0] = res_ref[0] + contrib


@jax.jit
def kernel(x, y, Wq, Wkv, Wproj, bproj, ln_w, ln_b, a1, a2):
    B, C, H, W = x.shape
    N = H * W
    hd = C // NUM_HEADS
    scale = hd ** (-0.5)
    kk1, kk2 = N // 2, N // 3

    pm = jnp.asarray(_pool_matrix(H))

    y_flat = y.reshape(B, C, N)
    x_flat = x.reshape(B, C, N)

    def _headT(wmat):
        wt = wmat.reshape(C, NUM_HEADS, hd).transpose(1, 2, 0)
        wt = jnp.pad(wt, ((0, 0), (0, HDP - hd), (0, 0)))
        return wt.reshape(NUM_HEADS * HDP, C)

    wkT = _headT(Wkv[:, :C])
    wvT = _headT(Wkv[:, C:])
    wqT = _headT(Wq)
    wpT = jnp.pad(Wproj.reshape(NUM_HEADS, hd, C),
                  ((0, 0), (0, HDP - hd), (0, 0))).transpose(0, 2, 1)

    kT, vT, qT = pl.pallas_call(
        _stage1_body,
        grid=(B,),
        in_specs=[
            pl.BlockSpec((1, C, N), lambda b: (b, 0, 0)),
            pl.BlockSpec((1, C, N), lambda b: (b, 0, 0)),
            pl.BlockSpec((N, N), lambda b: (0, 0)),
            pl.BlockSpec((NUM_HEADS * HDP, C), lambda b: (0, 0)),
            pl.BlockSpec((NUM_HEADS * HDP, C), lambda b: (0, 0)),
            pl.BlockSpec((NUM_HEADS * HDP, C), lambda b: (0, 0)),
            pl.BlockSpec((C, 1), lambda b: (0, 0)),
            pl.BlockSpec((C, 1), lambda b: (0, 0)),
        ],
        out_specs=[
            pl.BlockSpec((1, NUM_HEADS * HDP, N), lambda b: (b, 0, 0)),
            pl.BlockSpec((1, NUM_HEADS * HDP, N), lambda b: (b, 0, 0)),
            pl.BlockSpec((1, NUM_HEADS * HDP, N), lambda b: (b, 0, 0)),
        ],
        out_shape=[
            jax.ShapeDtypeStruct((B, NUM_HEADS * HDP, N), jnp.bfloat16),
            jax.ShapeDtypeStruct((B, NUM_HEADS * HDP, N), jnp.bfloat16),
            jax.ShapeDtypeStruct((B, NUM_HEADS * HDP, N), jnp.bfloat16),
        ],
    )(y_flat, x_flat, pm, wkT, wvT, wqT,
      ln_w.reshape(C, 1), ln_b.reshape(C, 1))

    resT = pl.pallas_call(
        functools.partial(_attn_body, kk1=kk1, kk2=kk2, scale=scale),
        grid=(B, NUM_HEADS),
        in_specs=[
            pl.BlockSpec((1, HDP, N), lambda b, h: (b, h, 0)),
            pl.BlockSpec((1, HDP, N), lambda b, h: (b, h, 0)),
            pl.BlockSpec((1, HDP, N), lambda b, h: (b, h, 0)),
            pl.BlockSpec((1, C, HDP), lambda b, h: (h, 0, 0)),
            pl.BlockSpec((C, 1), lambda b, h: (0, 0)),
            pl.BlockSpec((1, 1), lambda b, h: (0, 0)),
            pl.BlockSpec((1, 1), lambda b, h: (0, 0)),
        ],
        out_specs=pl.BlockSpec((1, C, N), lambda b, h: (b, 0, 0)),
        out_shape=jax.ShapeDtypeStruct((B, C, N), jnp.float32),
    )(qT, kT, vT, wpT, bproj.reshape(C, 1),
      a1.reshape(1, 1), a2.reshape(1, 1))

    return resT.reshape(B, C, H, W)
